# baseline (speedup 1.0000x reference)
import functools

import numpy as np
import jax
import jax.numpy as jnp
from jax import lax
from jax.experimental import pallas as pl
from jax.experimental.pallas import tpu as pltpu

B, E, S, D = 8, 8, 2048, 768
TOP_K = 2
CAPACITY = int(4.0 * B / E)
SLICE = S * D

GATE_KEY = (0, 1)
EXPERT_KEY = (0, 2)

_LO = np.nextafter(np.float32(-1.0), np.float32(0.0))
_SPAN = np.float32(np.float32(1.0) - _LO)
_LO2 = np.float32(_LO - _SPAN)

_ROTS = (13, 15, 26, 6, 17, 29, 16, 24)

_NORM_POLY = (1.253829836845398, -0.01309075765311718, 0.40694642066955566,
              -0.19156242907047272, 0.24166591465473175, -0.13486193120479584,
              0.035553522408008575, -0.004520737566053867,
              0.00022352249652612954)


def _threefry2x32(key, x0, x1):
    k0, k1 = key
    ks = (jnp.uint32(k0), jnp.uint32(k1),
          jnp.uint32(k0 ^ k1 ^ 0x1BD11BDA))
    x0 = x0 + ks[0]
    x1 = x1 + ks[1]
    for i in range(5):
        rots = _ROTS[:4] if i % 2 == 0 else _ROTS[4:]
        for r in rots:
            x0 = x0 + x1
            x1 = (x1 << r) | (x1 >> (32 - r))
            x1 = x0 ^ x1
        x0 = x0 + ks[(i + 1) % 3]
        x1 = x1 + ks[(i + 2) % 3] + jnp.uint32(i + 1)
    return x0, x1


def _random_bits(key, idx_u32):
    o0, o1 = _threefry2x32(key, jnp.zeros_like(idx_u32), idx_u32)
    return o0 ^ o1


def _bits_to_unit_float(bits):
    f = lax.bitcast_convert_type(
        (bits >> jnp.uint32(9)) | jnp.uint32(0x3F800000), jnp.float32)
    return f - jnp.float32(1.0)


def _normal_from_idx(idx_u32):
    bits = _random_bits(EXPERT_KEY, idx_u32)
    fb = lax.bitcast_convert_type(
        (bits >> jnp.uint32(9)) | jnp.uint32(0x3F800000), jnp.float32)
    x = (fb - jnp.float32(1.0)) * jnp.float32(_SPAN) + jnp.float32(_LO)
    s = jnp.sqrt(-jnp.log1p(-x * x))
    p = jnp.float32(_NORM_POLY[-1])
    for c in _NORM_POLY[-2::-1]:
        p = jnp.float32(c) + p * s
    return p * x


def _gating_body(sel_ref):
    row = lax.broadcasted_iota(jnp.int32, (B, E), 0)
    col = lax.broadcasted_iota(jnp.int32, (B, E), 1)
    bits = _random_bits(GATE_KEY, (row * E + col).astype(jnp.uint32))
    f = _bits_to_unit_float(bits)

    def top1(scores):
        m = jnp.max(scores, axis=1, keepdims=True)
        cand = jnp.where(scores == m, col, E)
        return jnp.min(cand, axis=1, keepdims=True)

    one = jnp.float32(1.0)
    zero = jnp.float32(0.0)
    e0 = top1(f)
    e1 = top1(jnp.where(col == e0, jnp.float32(-1.0), f))
    eq0 = jnp.where(col == e0, one, zero)
    eq1 = jnp.where(col == e1, one, zero)
    counts = (jnp.sum(eq0, axis=0, keepdims=True)
              + jnp.sum(eq1, axis=0, keepdims=True))
    cap_ok = jnp.where(counts < jnp.float32(CAPACITY), one, zero)
    c0 = jnp.sum(eq0 * cap_ok, axis=1, keepdims=True)
    c1 = jnp.sum(eq1 * cap_ok, axis=1, keepdims=True)
    p1 = jnp.where(c1 > jnp.float32(0.5), e1, E)
    sel0 = jnp.where(c0 > jnp.float32(0.5), e0, p1)
    sel1 = jnp.where(c0 > jnp.float32(0.5), p1, E)
    sel_ref[...] = jnp.concatenate([sel0, sel1], axis=1)


def _expert_sum_body(sel_ref, o_ref, *, rows):
    b = pl.program_id(0)
    j = pl.program_id(1)
    r = lax.broadcasted_iota(jnp.int32, (rows, D), 0)
    d = lax.broadcasted_iota(jnp.int32, (rows, D), 1)
    offs = r * D + d + j * (rows * D)
    e0 = sel_ref[b, 0]
    g = _normal_from_idx(((b * E + e0) * SLICE + offs).astype(jnp.uint32))
    o_ref[...] = jnp.where(e0 < E, g, jnp.float32(0.0))
    e1 = sel_ref[b, 1]

    @pl.when(e1 < E)
    def _():
        o_ref[...] += _normal_from_idx(
            ((b * E + e1) * SLICE + offs).astype(jnp.uint32))


def kernel(x):
    del x
    sel = pl.pallas_call(
        _gating_body,
        out_shape=jax.ShapeDtypeStruct((B, TOP_K), jnp.int32),
    )()

    rows = 2048
    nc = S // rows
    out = pl.pallas_call(
        functools.partial(_expert_sum_body, rows=rows),
        grid=(B, nc),
        in_specs=[pl.BlockSpec(memory_space=pltpu.MemorySpace.SMEM)],
        out_specs=pl.BlockSpec((rows, D), lambda b, j: (b * nc + j, 0)),
        out_shape=jax.ShapeDtypeStruct((B * S, D), jnp.float32),
        compiler_params=pltpu.CompilerParams(
            dimension_semantics=("parallel", "parallel")),
    )(sel)
    return out.reshape(B, S, D)


# speedup vs baseline: 1.2498x; 1.2498x over previous
import functools

import numpy as np
import jax
import jax.numpy as jnp
from jax import lax
from jax.experimental import pallas as pl
from jax.experimental.pallas import tpu as pltpu

B, E, S, D = 8, 8, 2048, 768
TOP_K = 2
CAPACITY = int(4.0 * B / E)
SLICE = S * D

GATE_KEY = (0, 1)
EXPERT_KEY = (0, 2)

_LO = np.nextafter(np.float32(-1.0), np.float32(0.0))
_SPAN = np.float32(np.float32(1.0) - _LO)
_LO2 = np.float32(_LO - _SPAN)

_ROTS = (13, 15, 26, 6, 17, 29, 16, 24)

_NORM_POLY = (1.253829836845398, -0.01309075765311718, 0.40694642066955566,
              -0.19156242907047272, 0.24166591465473175, -0.13486193120479584,
              0.035553522408008575, -0.004520737566053867,
              0.00022352249652612954)


def _threefry2x32(key, x0, x1):
    k0, k1 = key
    ks = (jnp.uint32(k0), jnp.uint32(k1),
          jnp.uint32(k0 ^ k1 ^ 0x1BD11BDA))
    x0 = x0 + ks[0]
    x1 = x1 + ks[1]
    for i in range(5):
        rots = _ROTS[:4] if i % 2 == 0 else _ROTS[4:]
        for r in rots:
            x0 = x0 + x1
            x1 = (x1 << r) | (x1 >> (32 - r))
            x1 = x0 ^ x1
        x0 = x0 + ks[(i + 1) % 3]
        x1 = x1 + ks[(i + 2) % 3] + jnp.uint32(i + 1)
    return x0, x1


def _random_bits(key, idx_u32):
    o0, o1 = _threefry2x32(key, jnp.zeros_like(idx_u32), idx_u32)
    return o0 ^ o1


def _bits_to_unit_float(bits):
    f = lax.bitcast_convert_type(
        (bits >> jnp.uint32(9)) | jnp.uint32(0x3F800000), jnp.float32)
    return f - jnp.float32(1.0)


def _normal_from_idx(idx_u32):
    bits = _random_bits(EXPERT_KEY, idx_u32)
    fb = lax.bitcast_convert_type(
        (bits >> jnp.uint32(9)) | jnp.uint32(0x3F800000), jnp.float32)
    x = (fb - jnp.float32(1.0)) * jnp.float32(_SPAN) + jnp.float32(_LO)
    s = jnp.sqrt(-jnp.log1p(-x * x))
    p = jnp.float32(_NORM_POLY[-1])
    for c in _NORM_POLY[-2::-1]:
        p = jnp.float32(c) + p * s
    return p * x


def _gating_body(sel_ref):
    row = lax.broadcasted_iota(jnp.int32, (B, E), 0)
    col = lax.broadcasted_iota(jnp.int32, (B, E), 1)
    bits = _random_bits(GATE_KEY, (row * E + col).astype(jnp.uint32))
    f = _bits_to_unit_float(bits)

    def top1(scores):
        m = jnp.max(scores, axis=1, keepdims=True)
        cand = jnp.where(scores == m, col, E)
        return jnp.min(cand, axis=1, keepdims=True)

    one = jnp.float32(1.0)
    zero = jnp.float32(0.0)
    e0 = top1(f)
    e1 = top1(jnp.where(col == e0, jnp.float32(-1.0), f))
    eq0 = jnp.where(col == e0, one, zero)
    eq1 = jnp.where(col == e1, one, zero)
    counts = (jnp.sum(eq0, axis=0, keepdims=True)
              + jnp.sum(eq1, axis=0, keepdims=True))
    cap_ok = jnp.where(counts < jnp.float32(CAPACITY), one, zero)
    c0 = jnp.sum(eq0 * cap_ok, axis=1, keepdims=True)
    c1 = jnp.sum(eq1 * cap_ok, axis=1, keepdims=True)
    p1 = jnp.where(c1 > jnp.float32(0.5), e1, E)
    sel0 = jnp.where(c0 > jnp.float32(0.5), e0, p1)
    sel1 = jnp.where(c0 > jnp.float32(0.5), p1, E)
    sel_ref[...] = jnp.concatenate([sel0, sel1], axis=1)


def _expert_sum_body(sel_ref, o_ref, *, rows):
    b = pl.program_id(0)
    j = pl.program_id(1)
    r = lax.broadcasted_iota(jnp.int32, (rows, D), 0)
    d = lax.broadcasted_iota(jnp.int32, (rows, D), 1)
    offs = r * D + d + j * (rows * D)
    e0 = sel_ref[b, 0]
    g = _normal_from_idx(((b * E + e0) * SLICE + offs).astype(jnp.uint32))
    o_ref[...] = jnp.where(e0 < E, g, jnp.float32(0.0))
    e1 = sel_ref[b, 1]

    @pl.when(e1 < E)
    def _():
        o_ref[...] += _normal_from_idx(
            ((b * E + e1) * SLICE + offs).astype(jnp.uint32))


def kernel(x):
    del x
    sel = pl.pallas_call(
        _gating_body,
        out_shape=jax.ShapeDtypeStruct((B, TOP_K), jnp.int32),
    )()

    rows = 512
    nc = S // rows
    out = pl.pallas_call(
        functools.partial(_expert_sum_body, rows=rows),
        grid=(B, nc),
        in_specs=[pl.BlockSpec(memory_space=pltpu.MemorySpace.SMEM)],
        out_specs=pl.BlockSpec((rows, D), lambda b, j: (b * nc + j, 0)),
        out_shape=jax.ShapeDtypeStruct((B * S, D), jnp.float32),
        compiler_params=pltpu.CompilerParams(
            dimension_semantics=("parallel", "parallel")),
    )(sel)
    return out.reshape(B, S, D)


# speedup vs baseline: 1.3402x; 1.0723x over previous
import functools

import numpy as np
import jax
import jax.numpy as jnp
from jax import lax
from jax.experimental import pallas as pl
from jax.experimental.pallas import tpu as pltpu

B, E, S, D = 8, 8, 2048, 768
TOP_K = 2
CAPACITY = int(4.0 * B / E)
SLICE = S * D

GATE_KEY = (0, 1)
EXPERT_KEY = (0, 2)

_LO = np.nextafter(np.float32(-1.0), np.float32(0.0))
_SPAN = np.float32(np.float32(1.0) - _LO)
_LO2 = np.float32(_LO - _SPAN)

_ROTS = (13, 15, 26, 6, 17, 29, 16, 24)

_NORM_POLY = (1.2507096529006958, 0.040698710829019547, 0.18401271104812622,
              0.18642151355743408, -0.07693851739168167, 0.009928256273269653,
              -0.00032247722265310585)


def _threefry2x32(key, x0, x1):
    k0, k1 = key
    ks = (jnp.uint32(k0), jnp.uint32(k1),
          jnp.uint32(k0 ^ k1 ^ 0x1BD11BDA))
    x0 = x0 + ks[0]
    x1 = x1 + ks[1]
    for i in range(5):
        rots = _ROTS[:4] if i % 2 == 0 else _ROTS[4:]
        for r in rots:
            x0 = x0 + x1
            x1 = (x1 << r) | (x1 >> (32 - r))
            x1 = x0 ^ x1
        x0 = x0 + ks[(i + 1) % 3]
        x1 = x1 + ks[(i + 2) % 3] + jnp.uint32(i + 1)
    return x0, x1


def _random_bits(key, idx_u32):
    o0, o1 = _threefry2x32(key, jnp.zeros_like(idx_u32), idx_u32)
    return o0 ^ o1


def _bits_to_unit_float(bits):
    f = lax.bitcast_convert_type(
        (bits >> jnp.uint32(9)) | jnp.uint32(0x3F800000), jnp.float32)
    return f - jnp.float32(1.0)


def _normal_from_idx(idx_u32):
    bits = _random_bits(EXPERT_KEY, idx_u32)
    fb = lax.bitcast_convert_type(
        (bits >> jnp.uint32(9)) | jnp.uint32(0x3F800000), jnp.float32)
    x = (fb - jnp.float32(1.0)) * jnp.float32(_SPAN) + jnp.float32(_LO)
    s = jnp.sqrt(-jnp.log(jnp.float32(1.0) - x * x))
    p = jnp.float32(_NORM_POLY[-1])
    for c in _NORM_POLY[-2::-1]:
        p = jnp.float32(c) + p * s
    return p * x


def _gating_body(sel_ref):
    row = lax.broadcasted_iota(jnp.int32, (B, E), 0)
    col = lax.broadcasted_iota(jnp.int32, (B, E), 1)
    bits = _random_bits(GATE_KEY, (row * E + col).astype(jnp.uint32))
    f = _bits_to_unit_float(bits)

    def top1(scores):
        m = jnp.max(scores, axis=1, keepdims=True)
        cand = jnp.where(scores == m, col, E)
        return jnp.min(cand, axis=1, keepdims=True)

    one = jnp.float32(1.0)
    zero = jnp.float32(0.0)
    e0 = top1(f)
    e1 = top1(jnp.where(col == e0, jnp.float32(-1.0), f))
    eq0 = jnp.where(col == e0, one, zero)
    eq1 = jnp.where(col == e1, one, zero)
    counts = (jnp.sum(eq0, axis=0, keepdims=True)
              + jnp.sum(eq1, axis=0, keepdims=True))
    cap_ok = jnp.where(counts < jnp.float32(CAPACITY), one, zero)
    c0 = jnp.sum(eq0 * cap_ok, axis=1, keepdims=True)
    c1 = jnp.sum(eq1 * cap_ok, axis=1, keepdims=True)
    p1 = jnp.where(c1 > jnp.float32(0.5), e1, E)
    sel0 = jnp.where(c0 > jnp.float32(0.5), e0, p1)
    sel1 = jnp.where(c0 > jnp.float32(0.5), p1, E)
    sel_ref[...] = jnp.concatenate([sel0, sel1], axis=1)


def _expert_sum_body(sel_ref, o_ref, *, rows):
    b = pl.program_id(0)
    j = pl.program_id(1)
    r = lax.broadcasted_iota(jnp.int32, (rows, D), 0)
    d = lax.broadcasted_iota(jnp.int32, (rows, D), 1)
    offs = r * D + d + j * (rows * D)
    e0 = sel_ref[b, 0]
    idx0 = ((b * E + e0) * SLICE + offs).astype(jnp.uint32)
    g = _normal_from_idx(idx0)
    o_ref[...] = jnp.where(e0 < E, g, jnp.float32(0.0))
    e1 = sel_ref[b, 1]

    @pl.when(e1 < E)
    def _():
        delta = ((e1 - e0) * SLICE).astype(jnp.uint32)
        o_ref[...] += _normal_from_idx(idx0 + delta)


def kernel(x):
    del x
    sel = pl.pallas_call(
        _gating_body,
        out_shape=jax.ShapeDtypeStruct((B, TOP_K), jnp.int32),
    )()

    rows = 512
    nc = S // rows
    out = pl.pallas_call(
        functools.partial(_expert_sum_body, rows=rows),
        grid=(B, nc),
        in_specs=[pl.BlockSpec(memory_space=pltpu.MemorySpace.SMEM)],
        out_specs=pl.BlockSpec((rows, D), lambda b, j: (b * nc + j, 0)),
        out_shape=jax.ShapeDtypeStruct((B * S, D), jnp.float32),
        compiler_params=pltpu.CompilerParams(
            dimension_semantics=("parallel", "parallel")),
    )(sel)
    return out.reshape(B, S, D)


# speedup vs baseline: 1.3681x; 1.0208x over previous
import functools

import numpy as np
import jax
import jax.numpy as jnp
from jax import lax
from jax.experimental import pallas as pl
from jax.experimental.pallas import tpu as pltpu

B, E, S, D = 8, 8, 2048, 768
TOP_K = 2
CAPACITY = int(4.0 * B / E)
SLICE = S * D

GATE_KEY = (0, 1)
EXPERT_KEY = (0, 2)

_LO = np.nextafter(np.float32(-1.0), np.float32(0.0))
_SPAN = np.float32(np.float32(1.0) - _LO)
_LO2 = np.float32(_LO - _SPAN)

_ROTS = (13, 15, 26, 6, 17, 29, 16, 24)

_NORM_POLY = (1.2514926195144653, 0.030848411843180656, 0.21269628405570984,
              0.15437479317188263, -0.06078895181417465, 0.006195908412337303)

_CVT_SCALE = np.float32(np.float64(_SPAN) * 2.0**-23)


def _threefry2x32(key, x0, x1):
    k0, k1 = key
    ks = (jnp.uint32(k0), jnp.uint32(k1),
          jnp.uint32(k0 ^ k1 ^ 0x1BD11BDA))
    x0 = x0 + ks[0]
    x1 = x1 + ks[1]
    for i in range(5):
        rots = _ROTS[:4] if i % 2 == 0 else _ROTS[4:]
        for r in rots:
            x0 = x0 + x1
            x1 = (x1 << r) | (x1 >> (32 - r))
            x1 = x0 ^ x1
        x0 = x0 + ks[(i + 1) % 3]
        x1 = x1 + ks[(i + 2) % 3] + jnp.uint32(i + 1)
    return x0, x1


def _random_bits(key, idx_u32):
    o0, o1 = _threefry2x32(key, jnp.zeros_like(idx_u32), idx_u32)
    return o0 ^ o1


def _bits_to_unit_float(bits):
    f = lax.bitcast_convert_type(
        (bits >> jnp.uint32(9)) | jnp.uint32(0x3F800000), jnp.float32)
    return f - jnp.float32(1.0)


def _normal_from_idx(idx_u32):
    bits = _random_bits(EXPERT_KEY, idx_u32)
    m = (bits >> jnp.uint32(9)).astype(jnp.int32)
    x = (m.astype(jnp.float32) * jnp.float32(_CVT_SCALE)
         + jnp.float32(_LO))
    s = jnp.sqrt(-jnp.log(jnp.float32(1.0) - x * x))
    p = jnp.float32(_NORM_POLY[-1])
    for c in _NORM_POLY[-2::-1]:
        p = jnp.float32(c) + p * s
    return p * x


def _gating_body(sel_ref):
    row = lax.broadcasted_iota(jnp.int32, (B, E), 0)
    col = lax.broadcasted_iota(jnp.int32, (B, E), 1)
    bits = _random_bits(GATE_KEY, (row * E + col).astype(jnp.uint32))
    f = _bits_to_unit_float(bits)

    def top1(scores):
        m = jnp.max(scores, axis=1, keepdims=True)
        cand = jnp.where(scores == m, col, E)
        return jnp.min(cand, axis=1, keepdims=True)

    one = jnp.float32(1.0)
    zero = jnp.float32(0.0)
    e0 = top1(f)
    e1 = top1(jnp.where(col == e0, jnp.float32(-1.0), f))
    eq0 = jnp.where(col == e0, one, zero)
    eq1 = jnp.where(col == e1, one, zero)
    counts = (jnp.sum(eq0, axis=0, keepdims=True)
              + jnp.sum(eq1, axis=0, keepdims=True))
    cap_ok = jnp.where(counts < jnp.float32(CAPACITY), one, zero)
    c0 = jnp.sum(eq0 * cap_ok, axis=1, keepdims=True)
    c1 = jnp.sum(eq1 * cap_ok, axis=1, keepdims=True)
    p1 = jnp.where(c1 > jnp.float32(0.5), e1, E)
    sel0 = jnp.where(c0 > jnp.float32(0.5), e0, p1)
    sel1 = jnp.where(c0 > jnp.float32(0.5), p1, E)
    sel_ref[...] = jnp.concatenate([sel0, sel1], axis=1)


def _expert_sum_body(sel_ref, o_ref, *, rows):
    b = pl.program_id(0)
    j = pl.program_id(1)
    r = lax.broadcasted_iota(jnp.int32, (rows, D), 0)
    d = lax.broadcasted_iota(jnp.int32, (rows, D), 1)
    offs = r * D + d + j * (rows * D)
    e0 = sel_ref[b, 0]
    idx0 = ((b * E + e0) * SLICE + offs).astype(jnp.uint32)
    g = _normal_from_idx(idx0)
    o_ref[...] = jnp.where(e0 < E, g, jnp.float32(0.0))
    e1 = sel_ref[b, 1]

    @pl.when(e1 < E)
    def _():
        delta = ((e1 - e0) * SLICE).astype(jnp.uint32)
        o_ref[...] += _normal_from_idx(idx0 + delta)


def kernel(x):
    del x
    sel = pl.pallas_call(
        _gating_body,
        out_shape=jax.ShapeDtypeStruct((B, TOP_K), jnp.int32),
    )()

    rows = 512
    nc = S // rows
    out = pl.pallas_call(
        functools.partial(_expert_sum_body, rows=rows),
        grid=(B, nc),
        in_specs=[pl.BlockSpec(memory_space=pltpu.MemorySpace.SMEM)],
        out_specs=pl.BlockSpec((rows, D), lambda b, j: (b * nc + j, 0)),
        out_shape=jax.ShapeDtypeStruct((B * S, D), jnp.float32),
        compiler_params=pltpu.CompilerParams(
            dimension_semantics=("parallel", "parallel")),
    )(sel)
    return out.reshape(B, S, D)


# speedup vs baseline: 1.3725x; 1.0032x over previous
import functools

import numpy as np
import jax
import jax.numpy as jnp
from jax import lax
from jax.experimental import pallas as pl
from jax.experimental.pallas import tpu as pltpu

B, E, S, D = 8, 8, 2048, 768
TOP_K = 2
CAPACITY = int(4.0 * B / E)
SLICE = S * D

GATE_KEY = (0, 1)
EXPERT_KEY = (0, 2)

_LO = np.nextafter(np.float32(-1.0), np.float32(0.0))
_SPAN = np.float32(np.float32(1.0) - _LO)
_LO2 = np.float32(_LO - _SPAN)

_ROTS = (13, 15, 26, 6, 17, 29, 16, 24)

_NORM_POLY = (1.2514926195144653, 0.030848411843180656, 0.21269628405570984,
              0.15437479317188263, -0.06078895181417465, 0.006195908412337303)

_CVT_SCALE = np.float32(np.float64(_SPAN) * 2.0**-23)


def _threefry2x32(key, x0, x1):
    k0, k1 = key
    ks = (jnp.uint32(k0), jnp.uint32(k1),
          jnp.uint32(k0 ^ k1 ^ 0x1BD11BDA))
    x0 = x0 + ks[0]
    x1 = x1 + ks[1]
    for i in range(5):
        rots = _ROTS[:4] if i % 2 == 0 else _ROTS[4:]
        for r in rots:
            x0 = x0 + x1
            x1 = (x1 << r) | (x1 >> (32 - r))
            x1 = x0 ^ x1
        x0 = x0 + ks[(i + 1) % 3]
        x1 = x1 + ks[(i + 2) % 3] + jnp.uint32(i + 1)
    return x0, x1


def _random_bits(key, idx_u32):
    o0, o1 = _threefry2x32(key, jnp.zeros_like(idx_u32), idx_u32)
    return o0 ^ o1


def _bits_to_unit_float(bits):
    f = lax.bitcast_convert_type(
        (bits >> jnp.uint32(9)) | jnp.uint32(0x3F800000), jnp.float32)
    return f - jnp.float32(1.0)


def _normal_from_idx(idx_u32):
    bits = _random_bits(EXPERT_KEY, idx_u32)
    m = (bits >> jnp.uint32(9)).astype(jnp.int32)
    x = (m.astype(jnp.float32) * jnp.float32(_CVT_SCALE)
         + jnp.float32(_LO))
    s = jnp.sqrt(-jnp.log(jnp.float32(1.0) - x * x))
    p = jnp.float32(_NORM_POLY[-1])
    for c in _NORM_POLY[-2::-1]:
        p = jnp.float32(c) + p * s
    return p * x


def _gating_body(sel_ref):
    row = lax.broadcasted_iota(jnp.int32, (B, E), 0)
    col = lax.broadcasted_iota(jnp.int32, (B, E), 1)
    bits = _random_bits(GATE_KEY, (row * E + col).astype(jnp.uint32))
    f = _bits_to_unit_float(bits)

    def top1(scores):
        m = jnp.max(scores, axis=1, keepdims=True)
        cand = jnp.where(scores == m, col, E)
        return jnp.min(cand, axis=1, keepdims=True)

    one = jnp.float32(1.0)
    zero = jnp.float32(0.0)
    e0 = top1(f)
    e1 = top1(jnp.where(col == e0, jnp.float32(-1.0), f))
    eq0 = jnp.where(col == e0, one, zero)
    eq1 = jnp.where(col == e1, one, zero)
    counts = (jnp.sum(eq0, axis=0, keepdims=True)
              + jnp.sum(eq1, axis=0, keepdims=True))
    cap_ok = jnp.where(counts < jnp.float32(CAPACITY), one, zero)
    c0 = jnp.sum(eq0 * cap_ok, axis=1, keepdims=True)
    c1 = jnp.sum(eq1 * cap_ok, axis=1, keepdims=True)
    p1 = jnp.where(c1 > jnp.float32(0.5), e1, E)
    sel0 = jnp.where(c0 > jnp.float32(0.5), e0, p1)
    sel1 = jnp.where(c0 > jnp.float32(0.5), p1, E)
    sel_ref[...] = jnp.concatenate([sel0, sel1], axis=1)


def _expert_sum_body(sel_ref, o_ref, *, rows):
    b = pl.program_id(0)
    j = pl.program_id(1)
    r = lax.broadcasted_iota(jnp.int32, (rows, D), 0)
    d = lax.broadcasted_iota(jnp.int32, (rows, D), 1)
    offs = r * D + d + j * (rows * D)
    e0 = sel_ref[b, 0]
    idx0 = ((b * E + e0) * SLICE + offs).astype(jnp.uint32)
    g = _normal_from_idx(idx0)
    o_ref[...] = jnp.where(e0 < E, g, jnp.float32(0.0))
    e1 = sel_ref[b, 1]

    @pl.when(e1 < E)
    def _():
        delta = ((e1 - e0) * SLICE).astype(jnp.uint32)
        o_ref[...] += _normal_from_idx(idx0 + delta)


def kernel(x):
    del x
    sel = pl.pallas_call(
        _gating_body,
        out_shape=jax.ShapeDtypeStruct((B, TOP_K), jnp.int32),
    )()

    rows = 1024
    nc = S // rows
    out = pl.pallas_call(
        functools.partial(_expert_sum_body, rows=rows),
        grid=(B, nc),
        in_specs=[pl.BlockSpec(memory_space=pltpu.MemorySpace.SMEM)],
        out_specs=pl.BlockSpec((rows, D), lambda b, j: (b * nc + j, 0)),
        out_shape=jax.ShapeDtypeStruct((B * S, D), jnp.float32),
        compiler_params=pltpu.CompilerParams(
            dimension_semantics=("parallel", "parallel")),
    )(sel)
    return out.reshape(B, S, D)


# speedup vs baseline: 1.3756x; 1.0023x over previous
import functools

import numpy as np
import jax
import jax.numpy as jnp
from jax import lax
from jax.experimental import pallas as pl
from jax.experimental.pallas import tpu as pltpu

B, E, S, D = 8, 8, 2048, 768
TOP_K = 2
CAPACITY = int(4.0 * B / E)
SLICE = S * D

GATE_KEY = (0, 1)
EXPERT_KEY = (0, 2)

_LO = np.nextafter(np.float32(-1.0), np.float32(0.0))
_SPAN = np.float32(np.float32(1.0) - _LO)
_LO2 = np.float32(_LO - _SPAN)

_ROTS = (13, 15, 26, 6, 17, 29, 16, 24)

_NORM_POLY = (1.2514926195144653, 0.030848411843180656, 0.21269628405570984,
              0.15437479317188263, -0.06078895181417465, 0.006195908412337303)

_CVT_SCALE = np.float32(np.float64(_SPAN) * 2.0**-23)


def _threefry2x32(key, x0, x1):
    k0, k1 = key
    ks = (jnp.uint32(k0), jnp.uint32(k1),
          jnp.uint32(k0 ^ k1 ^ 0x1BD11BDA))
    x0 = x0 + ks[0]
    x1 = x1 + ks[1]
    for i in range(5):
        rots = _ROTS[:4] if i % 2 == 0 else _ROTS[4:]
        for r in rots:
            x0 = x0 + x1
            x1 = (x1 << r) | (x1 >> (32 - r))
            x1 = x0 ^ x1
        x0 = x0 + ks[(i + 1) % 3]
        x1 = x1 + ks[(i + 2) % 3] + jnp.uint32(i + 1)
    return x0, x1


def _random_bits(key, idx_u32):
    o0, o1 = _threefry2x32(key, jnp.zeros_like(idx_u32), idx_u32)
    return o0 ^ o1


def _bits_to_unit_float(bits):
    f = lax.bitcast_convert_type(
        (bits >> jnp.uint32(9)) | jnp.uint32(0x3F800000), jnp.float32)
    return f - jnp.float32(1.0)


def _normal_from_idx(idx_u32):
    bits = _random_bits(EXPERT_KEY, idx_u32)
    m = (bits >> jnp.uint32(9)).astype(jnp.int32)
    x = (m.astype(jnp.float32) * jnp.float32(_CVT_SCALE)
         + jnp.float32(_LO))
    s = jnp.sqrt(-jnp.log(jnp.float32(1.0) - x * x))
    p = jnp.float32(_NORM_POLY[-1])
    for c in _NORM_POLY[-2::-1]:
        p = jnp.float32(c) + p * s
    return p * x


def _gating_body(sel_ref):
    row = lax.broadcasted_iota(jnp.int32, (B, E), 0)
    col = lax.broadcasted_iota(jnp.int32, (B, E), 1)
    bits = _random_bits(GATE_KEY, (row * E + col).astype(jnp.uint32))
    f = _bits_to_unit_float(bits)

    def top1(scores):
        m = jnp.max(scores, axis=1, keepdims=True)
        cand = jnp.where(scores == m, col, E)
        return jnp.min(cand, axis=1, keepdims=True)

    one = jnp.float32(1.0)
    zero = jnp.float32(0.0)
    e0 = top1(f)
    e1 = top1(jnp.where(col == e0, jnp.float32(-1.0), f))
    eq0 = jnp.where(col == e0, one, zero)
    eq1 = jnp.where(col == e1, one, zero)
    counts = (jnp.sum(eq0, axis=0, keepdims=True)
              + jnp.sum(eq1, axis=0, keepdims=True))
    cap_ok = jnp.where(counts < jnp.float32(CAPACITY), one, zero)
    c0 = jnp.sum(eq0 * cap_ok, axis=1, keepdims=True)
    c1 = jnp.sum(eq1 * cap_ok, axis=1, keepdims=True)
    p1 = jnp.where(c1 > jnp.float32(0.5), e1, E)
    sel0 = jnp.where(c0 > jnp.float32(0.5), e0, p1)
    sel1 = jnp.where(c0 > jnp.float32(0.5), p1, E)
    sel_ref[...] = jnp.concatenate([sel0, sel1], axis=1)


def _expert_sum_body(sel_ref, offs_ref, o_ref, *, rows):
    b = pl.program_id(0)
    j = pl.program_id(1)
    e0 = sel_ref[b, 0]
    base = (b * E + e0) * SLICE + j * (rows * D)
    idx0 = (base + offs_ref[...]).astype(jnp.uint32)
    g = _normal_from_idx(idx0)
    o_ref[...] = jnp.where(e0 < E, g, jnp.float32(0.0))
    e1 = sel_ref[b, 1]

    @pl.when(e1 < E)
    def _():
        delta = ((e1 - e0) * SLICE).astype(jnp.uint32)
        o_ref[...] += _normal_from_idx(idx0 + delta)


def kernel(x):
    del x
    sel = pl.pallas_call(
        _gating_body,
        out_shape=jax.ShapeDtypeStruct((B, TOP_K), jnp.int32),
    )()

    rows = 1024
    nc = S // rows
    offs = jnp.arange(rows * D, dtype=jnp.int32).reshape(rows, D)
    out = pl.pallas_call(
        functools.partial(_expert_sum_body, rows=rows),
        grid=(B, nc),
        in_specs=[pl.BlockSpec(memory_space=pltpu.MemorySpace.SMEM),
                  pl.BlockSpec((rows, D), lambda b, j: (0, 0))],
        out_specs=pl.BlockSpec((rows, D), lambda b, j: (b * nc + j, 0)),
        out_shape=jax.ShapeDtypeStruct((B * S, D), jnp.float32),
        compiler_params=pltpu.CompilerParams(
            dimension_semantics=("parallel", "parallel")),
    )(sel, offs)
    return out.reshape(B, S, D)
